# baseline (speedup 1.0000x reference)
import jax
import jax.numpy as jnp
from jax.experimental import pallas as pl
from jax.experimental.pallas import tpu as pltpu

CAP = 1_000_000
BLK = 8192
NB = (CAP + BLK - 1) // BLK
NEG = -3.0e38
IBIG = 2**31 - 1


def _body(idx_ref, phs_ref, pm_ref, trajT_ref, bufT_ref, bufT_any, out_ref,
          gmax_sm, gidx_sm, bm_sm, wrow_ref, sem):
    b = pl.program_id(0)
    idx = idx_ref[0]

    @pl.when(b == 0)
    def _init():
        gmax_sm[0] = NEG
        gidx_sm[0] = IBIG

    bf = bufT_ref[0:4, :].astype(jnp.bfloat16).astype(jnp.float32)
    out8 = jax.lax.dot_general(
        pm_ref[...], bf, (((1,), (0,)), ((), ())),
        precision=jax.lax.Precision.HIGHEST,
        preferred_element_type=jnp.float32)
    bm_sm[0] = jnp.max(out8)

    special = (b == idx // BLK) | (b == NB - 1)

    @pl.when(special)
    def _masked_max():
        col = jax.lax.broadcasted_iota(jnp.int32, (8, BLK), 1)
        bad = (col == idx - b * BLK) | (col >= CAP - b * BLK)
        bm_sm[0] = jnp.max(jnp.where(bad, NEG, out8))

    bmax = bm_sm[0]

    @pl.when(bmax > gmax_sm[0])
    def _win():
        col = jax.lax.broadcasted_iota(jnp.int32, (8, BLK), 1)
        bad = (col == idx - b * BLK) | (col >= CAP - b * BLK)
        sm = jnp.where(bad, NEG, out8)
        ja = jnp.min(jnp.where(sm == bmax, col, IBIG))
        gmax_sm[0] = bmax
        gidx_sm[0] = b * BLK + ja

    @pl.when(b == NB - 1)
    def _finish():
        gi = gidx_sm[0]
        j0 = pl.multiple_of((gi // 128) * 128, 128)
        cp = pltpu.make_async_copy(bufT_any.at[:, pl.ds(j0, 128)], wrow_ref, sem)
        cp.start()
        cp.wait()
        lane = gi - j0
        colw = jax.lax.broadcasted_iota(jnp.int32, (8, 128), 1)
        w = jnp.where(colw == lane, wrow_ref[...], 0.0)
        roww = jnp.sum(w, axis=1, keepdims=True)
        row_act = roww[4:7, :]

        asum = jnp.sum(trajT_ref[...], axis=1, keepdims=True)
        theta = jnp.sqrt(jnp.sum(asum * asum))
        axis = asum / (theta + 1e-8)
        qr = jnp.cos(theta)
        qi = axis * jnp.sin(theta)
        to_f = lambda x: x.astype(jnp.bfloat16).astype(jnp.float32)
        sim_e = (to_f(qr) * phs_ref[0] + to_f(qi[0, 0]) * phs_ref[1]
                 + to_f(qi[1, 0]) * phs_ref[2] + to_f(qi[2, 0]) * phs_ref[3])
        gmax = gmax_sm[0]
        win_e = (sim_e > gmax) | ((sim_e == gmax) & (idx < gidx_sm[0]))

        res = jnp.where(win_e, asum, row_act)
        out_ref[...] = jnp.broadcast_to(res, (3, 128))


def kernel(trajectory_lie_elements, value, current_phase, buffer, ptr):
    del value
    idx = (jnp.asarray(ptr, jnp.int32) % CAP).reshape(1)
    bufT = buffer.T
    trajT = trajectory_lie_elements.T
    pm = jnp.broadcast_to(current_phase.reshape(1, 4), (8, 4))

    out = pl.pallas_call(
        _body,
        grid=(NB,),
        in_specs=[
            pl.BlockSpec(memory_space=pltpu.SMEM),
            pl.BlockSpec(memory_space=pltpu.SMEM),
            pl.BlockSpec((8, 4), lambda b: (0, 0)),
            pl.BlockSpec((3, 8192), lambda b: (0, 0)),
            pl.BlockSpec((8, BLK), lambda b: (0, b)),
            pl.BlockSpec(memory_space=pl.ANY),
        ],
        out_specs=pl.BlockSpec((3, 128), lambda b: (0, 0)),
        out_shape=jax.ShapeDtypeStruct((3, 128), jnp.float32),
        scratch_shapes=[
            pltpu.SMEM((1,), jnp.float32),
            pltpu.SMEM((1,), jnp.int32),
            pltpu.SMEM((1,), jnp.float32),
            pltpu.VMEM((8, 128), jnp.float32),
            pltpu.SemaphoreType.DMA,
        ],
        compiler_params=pltpu.CompilerParams(
            dimension_semantics=("arbitrary",),
        ),
    )(idx, current_phase, pm, trajT, bufT, bufT)
    return out[:, 0]


# speedup vs baseline: 1.4479x; 1.4479x over previous
import jax
import jax.numpy as jnp
from jax.experimental import pallas as pl
from jax.experimental.pallas import tpu as pltpu

CAP = 1_000_000
BLK = 8192
NB = (CAP + BLK - 1) // BLK
NEG = -3.0e38
IBIG = 2**31 - 1

PADCAP = ((CAP + 127) // 128) * 128
LAST_BASE = PADCAP - BLK


def _base(blk):
    return jnp.where(blk == NB - 1, LAST_BASE, blk * BLK)


def _dma(bufT_any, dbuf, sems, blk, slot):
    off = pl.multiple_of(_base(blk), 128)
    return pltpu.make_async_copy(
        bufT_any.at[pl.ds(0, 4), pl.ds(off, BLK)],
        dbuf.at[slot], sems.at[slot])


def _sims_of(blk4, ph):
    bf = blk4.astype(jnp.bfloat16).astype(jnp.float32)
    prod = bf * ph
    psum = prod[0:2, :] + prod[2:4, :]
    return psum[0:1, :] + psum[1:2, :]


def _grpmax(sims):
    parts = [sims[:, k * 128:(k + 1) * 128] for k in range(BLK // 128)]
    while len(parts) > 1:
        parts = [jnp.maximum(parts[i], parts[i + 1])
                 for i in range(0, len(parts), 2)]
    return parts[0]


def _body(idx_ref, phs_ref, ph_ref, trajT_ref, bufT_any, out_ref,
          rmax_ref, rbid_ref, sc_sm, dbuf, sems, wrow_ref, sem):
    b = pl.program_id(0)
    idx = idx_ref[0]

    @pl.when(b == 0)
    def _init():
        rmax_ref[...] = jnp.full((1, 128), NEG, jnp.float32)
        rbid_ref[...] = jnp.zeros((1, 128), jnp.int32)
        _dma(bufT_any, dbuf, sems, 0, 0).start()

    @pl.when(b + 1 < NB)
    def _prefetch():
        _dma(bufT_any, dbuf, sems, b + 1, (b + 1) % 2).start()

    _dma(bufT_any, dbuf, sems, b, b % 2).wait()
    blk4 = dbuf[b % 2]
    sims = _sims_of(blk4, ph_ref[...])
    base = _base(b)
    special = (b == idx // BLK) | (b == NB - 1)

    def _update(s):
        m128 = _grpmax(s)
        upd = m128 > rmax_ref[...]
        rmax_ref[...] = jnp.where(upd, m128, rmax_ref[...])
        rbid_ref[...] = jnp.where(upd, b, rbid_ref[...])

    @pl.when(jnp.logical_not(special))
    def _plain():
        _update(sims)

    @pl.when(special)
    def _masked():
        col = jax.lax.broadcasted_iota(jnp.int32, (1, BLK), 1)
        bad = (col == idx - base) | (col >= CAP - base)
        _update(jnp.where(bad, NEG, sims))

    @pl.when(b == NB - 1)
    def _finish():
        rmax = rmax_ref[...]
        gmax = jnp.max(rmax)
        bs = jnp.min(jnp.where(rmax == gmax, rbid_ref[...], IBIG))
        sc_sm[0] = bs
        bsc = sc_sm[0]
        _dma(bufT_any, dbuf, sems, bsc, 0).start()
        _dma(bufT_any, dbuf, sems, bsc, 0).wait()
        base2 = _base(bsc)
        sims2 = _sims_of(dbuf[0], ph_ref[...])
        col = jax.lax.broadcasted_iota(jnp.int32, (1, BLK), 1)
        bad = (col == idx - base2) | (col >= CAP - base2)
        sm = jnp.where(bad, NEG, sims2)
        ja = jnp.min(jnp.where(sm == gmax, col, IBIG))
        sc_sm[1] = base2 + ja
        gi = sc_sm[1]

        j0 = pl.multiple_of((gi // 128) * 128, 128)
        cp = pltpu.make_async_copy(bufT_any.at[:, pl.ds(j0, 128)], wrow_ref, sem)
        cp.start()
        cp.wait()
        colw = jax.lax.broadcasted_iota(jnp.int32, (8, 128), 1)
        w = jnp.where(colw == gi - j0, wrow_ref[...], 0.0)
        roww = jnp.sum(w, axis=1, keepdims=True)
        row_act = roww[4:7, :]

        asum = jnp.sum(trajT_ref[...], axis=1, keepdims=True)
        theta = jnp.sqrt(jnp.sum(asum * asum))
        axis = asum / (theta + 1e-8)
        qr = jnp.cos(theta)
        qi = axis * jnp.sin(theta)
        to_f = lambda x: x.astype(jnp.bfloat16).astype(jnp.float32)
        sim_e = (to_f(qr) * phs_ref[0] + to_f(qi[0, 0]) * phs_ref[1]
                 + to_f(qi[1, 0]) * phs_ref[2] + to_f(qi[2, 0]) * phs_ref[3])
        win_e = (sim_e > gmax) | ((sim_e == gmax) & (idx < gi))

        res = jnp.where(win_e, asum, row_act)
        out_ref[...] = jnp.broadcast_to(res, (3, 128))


def kernel(trajectory_lie_elements, value, current_phase, buffer, ptr):
    del value
    idx = (jnp.asarray(ptr, jnp.int32) % CAP).reshape(1)
    bufT = buffer.T
    trajT = trajectory_lie_elements.T
    ph_col = current_phase.reshape(4, 1)

    out = pl.pallas_call(
        _body,
        grid=(NB,),
        in_specs=[
            pl.BlockSpec(memory_space=pltpu.SMEM),
            pl.BlockSpec(memory_space=pltpu.SMEM),
            pl.BlockSpec((4, 1), lambda b: (0, 0)),
            pl.BlockSpec((3, 8192), lambda b: (0, 0)),
            pl.BlockSpec(memory_space=pl.ANY),
        ],
        out_specs=pl.BlockSpec((3, 128), lambda b: (0, 0)),
        out_shape=jax.ShapeDtypeStruct((3, 128), jnp.float32),
        scratch_shapes=[
            pltpu.VMEM((1, 128), jnp.float32),
            pltpu.VMEM((1, 128), jnp.int32),
            pltpu.SMEM((2,), jnp.int32),
            pltpu.VMEM((2, 4, BLK), jnp.float32),
            pltpu.SemaphoreType.DMA((2,)),
            pltpu.VMEM((8, 128), jnp.float32),
            pltpu.SemaphoreType.DMA,
        ],
        compiler_params=pltpu.CompilerParams(
            dimension_semantics=("arbitrary",),
        ),
    )(idx, current_phase, ph_col, trajT, bufT)
    return out[:, 0]
